# baseline (speedup 1.0000x reference)
import functools

import jax
import jax.numpy as jnp
from jax import lax
from jax.experimental import pallas as pl
from jax.experimental.pallas import tpu as pltpu
from jax.experimental.pallas import tpu_sc as plsc

R = 3
E_CONSTANT = 2.7182817459106445
BN_EPS = 1e-5
F32 = jnp.float32

_NC = 2
_NS = 16
_CH = 120


def _mm_stats_body(n, x_ref, age_ref, w_ref, y_ref, stats_ref):
    i = pl.program_id(0)
    y = jnp.dot(x_ref[...], w_ref[...], preferred_element_type=F32)
    y_ref[...] = y
    age = age_ref[...].astype(F32)
    rows = []
    for w in range(R):
        m = (age >= float(w)).astype(F32)
        ym = y * m
        rows.append(jnp.sum(ym, axis=0))
        rows.append(jnp.sum(ym * ym, axis=0))
    z = jnp.zeros_like(rows[0])
    stats = jnp.stack(rows + [z, z])

    @pl.when(i == 0)
    def _():
        stats_ref[...] = stats

    @pl.when(i != 0)
    def _():
        stats_ref[...] += stats


def _apply1_body(n, y_ref, age_ref, stats_ref, h0_ref, h1_ref, h2_ref):
    y = y_ref[...]
    age = age_ref[...].astype(F32)
    stats = stats_ref[...]
    outs = (h0_ref, h1_ref, h2_ref)
    for w in range(R):
        s = stats[2 * w]
        q = stats[2 * w + 1]
        m = s / n
        inv = lax.rsqrt(q / n - m * m + BN_EPS)
        msk = (age >= float(w)).astype(F32)
        outs[w][...] = jnp.maximum((y * msk - m[None, :]) * inv[None, :], 0.0)


def _z_stats_body(n, p0_ref, p1_ref, p2_ref, w_ref, b_ref,
                  z0_ref, z1_ref, z2_ref, stats_ref):
    i = pl.program_id(0)
    wmat = w_ref[...]
    b = b_ref[...]
    rows = []
    for p_ref, z_ref in ((p0_ref, z0_ref), (p1_ref, z1_ref), (p2_ref, z2_ref)):
        agg = p_ref[0] + p_ref[1]
        z = jnp.dot(agg, wmat, preferred_element_type=F32) + b
        z_ref[...] = z
        rows.append(jnp.sum(z, axis=0))
        rows.append(jnp.sum(z * z, axis=0))
    zr = jnp.zeros_like(rows[0])
    stats = jnp.stack(rows + [zr, zr])

    @pl.when(i == 0)
    def _():
        stats_ref[...] = stats

    @pl.when(i != 0)
    def _():
        stats_ref[...] += stats


def _apply_bn_body(n, z0_ref, z1_ref, z2_ref, stats_ref, h0_ref, h1_ref, h2_ref):
    stats = stats_ref[...]
    zs = (z0_ref, z1_ref, z2_ref)
    outs = (h0_ref, h1_ref, h2_ref)
    for w in range(R):
        s = stats[2 * w]
        q = stats[2 * w + 1]
        m = s / n
        inv = lax.rsqrt(q / n - m * m + BN_EPS)
        outs[w][...] = jnp.maximum((zs[w][...] - m[None, :]) * inv[None, :], 0.0)


def _final_body(n, h1_0, h1_1, h1_2, h2_0, h2_1, h2_2,
                z3_0, z3_1, z3_2, stats_ref, out_ref, nsum_ref):
    i = pl.program_id(0)
    stats = stats_ref[...]
    h1s = (h1_0, h1_1, h1_2)
    h2s = (h2_0, h2_1, h2_2)
    z3s = (z3_0, z3_1, z3_2)
    wins = []
    for w in range(R):
        s = stats[2 * w]
        q = stats[2 * w + 1]
        m = s / n
        inv = lax.rsqrt(q / n - m * m + BN_EPS)
        h3 = jnp.maximum((z3s[w][...] - m[None, :]) * inv[None, :], 0.0)
        wins.append((h1s[w][...], h2s[w][...], h3))
    normsq = None
    for j in range(3):
        a0 = wins[0][j]
        t = a0 - 0.5 * (wins[1][j] + wins[2][j])
        out_ref[:, j * 128:(j + 1) * 128] = a0
        out_ref[:, 384 + j * 128:384 + (j + 1) * 128] = t
        nsq = jnp.sum(t * t, axis=1)
        normsq = nsq if normsq is None else normsq + nsq
    total = jnp.sum(jnp.sqrt(normsq)).reshape(1, 1)

    @pl.when(i == 0)
    def _():
        nsum_ref[...] = total

    @pl.when(i != 0)
    def _():
        nsum_ref[...] += total


def _make_hop(n_nodes, e_pad, r_acc, nch0, nch1):
    rows_per_tile = r_acc // _NS
    mesh = plsc.VectorSubcoreMesh(core_axis_name="c", subcore_axis_name="s")
    out_sd = jax.ShapeDtypeStruct((_NC, r_acc, 128), F32)

    @functools.partial(
        pl.kernel,
        mesh=mesh,
        out_type=(out_sd, out_sd, out_sd),
        scratch_types=[
            pltpu.VMEM_SHARED((r_acc, 128), F32),
            pltpu.VMEM((6, 2, _CH), jnp.int32),
            pltpu.VMEM((3, _CH, 128), F32),
            pltpu.SemaphoreType.DMA((3,)),
            pltpu.SemaphoreType.DMA((3,)),
            pltpu.SemaphoreType.DMA((6,)),
        ],
    )
    def hop(t0, t1, t2, idx_hbm, o0, o1, o2,
            acc, idxb, rowb, gsem, ssem, isem):
        c = lax.axis_index("c")
        s = lax.axis_index("s")
        tile_row0 = s * rows_per_tile
        nch = jnp.where(c == 0, nch0, nch1)
        chunk0 = jnp.where(c == 0, s * nch0, _NS * nch0 + s * nch1)

        def _ld_idx(i, slot):
            return pltpu.make_async_copy(idx_hbm.at[chunk0 + i],
                                         idxb.at[slot], isem.at[slot])

        def _gather(tbl, i6, b3):
            return pltpu.make_async_copy(tbl.at[idxb.at[i6, 0]],
                                         rowb.at[b3], gsem.at[b3])

        def _scatter(i6, b3):
            return pltpu.make_async_copy(rowb.at[b3],
                                         acc.at[idxb.at[i6, 1]],
                                         ssem.at[b3])

        for w in range(R):
            tbl = (t0, t1, t2)[w]
            out = (o0, o1, o2)[w]

            def _zb(i, carry):
                for j in range(128 // 16):
                    rowb[0, i, pl.ds(j * 16, 16)] = jnp.zeros((16,), F32)
                return carry

            lax.fori_loop(0, _CH, _zb, 0)
            done = 0
            while done < rows_per_tile:
                rows = min(_CH, rows_per_tile - done)
                pltpu.sync_copy(rowb.at[0, pl.ds(0, rows)],
                                acc.at[pl.ds(tile_row0 + done, rows)])
                done += rows
            plsc.subcore_barrier()

            for k in range(4):
                _ld_idx(k, k).start()
            _ld_idx(0, 0).wait()
            _ld_idx(1, 1).wait()
            _gather(tbl, 0, 0).start()
            _gather(tbl, 1, 1).start()

            def _grp(g, carry):
                for b in range(6):
                    i = 6 * g + b
                    b3 = b % 3
                    _gather(tbl, b, b3).wait()

                    @pl.when(i >= 1)
                    def _():
                        _scatter((b + 5) % 6, (b3 + 2) % 3).wait()

                    _scatter(b, b3).start(add=True)

                    @pl.when(i + 4 < nch)
                    def _():
                        _ld_idx(i + 4, (b + 4) % 6).start()

                    @pl.when(i + 2 < nch)
                    def _():
                        _ld_idx(i + 2, (b + 2) % 6).wait()
                        _gather(tbl, (b + 2) % 6, (b3 + 2) % 3).start()
                return carry

            lax.fori_loop(0, nch // 6, _grp, 0)
            _scatter(5, 2).wait()
            plsc.subcore_barrier()

            pltpu.sync_copy(acc.at[pl.ds(tile_row0, rows_per_tile)],
                            out.at[c, pl.ds(tile_row0, rows_per_tile)])
            plsc.subcore_barrier()

    return hop


def _cdiv(a, b):
    return (a + b - 1) // b


def kernel(x, edge_index, age, init_weight, W0, b0, W1, b1):
    n, d = x.shape
    h = init_weight.shape[1]
    e = edge_index.shape[1]
    bn = 1000
    grid = n // bn
    nf = float(n)

    e_unit = _NC * _NS * 6 * _CH
    e_pad = _cdiv(e, e_unit) * e_unit
    r_acc = _cdiv(n, _NS * 8) * (_NS * 8)
    if r_acc == n:
        r_acc += _NS * 8
    pad = e_pad - e
    dummy = n + jnp.arange(pad, dtype=jnp.int32) % (r_acc - n)
    src = jnp.concatenate([edge_index[0], jnp.zeros((pad,), jnp.int32)])
    dst = jnp.concatenate([edge_index[1], dummy])
    idx3 = jnp.stack([src.reshape(-1, _CH), dst.reshape(-1, _CH)], axis=1)
    age2 = age.reshape(n, 1)

    row_spec = pl.BlockSpec((bn, h), lambda i: (i, 0))
    age_spec = pl.BlockSpec((bn, 1), lambda i: (i, 0))
    stats_spec = pl.BlockSpec((8, h), lambda i: (0, 0))
    full_spec = pl.BlockSpec((d, h), lambda i: (0, 0))
    bias_spec = pl.BlockSpec((1, h), lambda i: (0, 0))
    part_spec = pl.BlockSpec((_NC, bn, h), lambda i: (0, i, 0))
    out_spec = pl.BlockSpec((bn, 6 * h), lambda i: (i, 0))
    scal_spec = pl.BlockSpec((1, 1), lambda i: (0, 0))

    row_sd = jax.ShapeDtypeStruct((n, h), F32)
    stats_sd = jax.ShapeDtypeStruct((8, h), F32)

    y, stats1 = pl.pallas_call(
        functools.partial(_mm_stats_body, nf),
        grid=(grid,),
        in_specs=[row_spec, age_spec, full_spec],
        out_specs=[row_spec, stats_spec],
        out_shape=[row_sd, stats_sd],
    )(x, age2, init_weight)

    h1 = pl.pallas_call(
        functools.partial(_apply1_body, nf),
        grid=(grid,),
        in_specs=[row_spec, age_spec, stats_spec],
        out_specs=[row_spec] * 3,
        out_shape=[row_sd] * 3,
    )(y, age2, stats1)

    nch_tot = e_pad // (_NS * _CH)
    nch0 = (int(round(nch_tot * 0.964)) // 6) * 6
    nch1 = nch_tot - nch0
    hop = _make_hop(n, e_pad, r_acc, nch0, nch1)

    def dense_hop(tables, wmat, bias):
        parts = hop(tables[0], tables[1], tables[2], idx3)
        z_and_stats = pl.pallas_call(
            functools.partial(_z_stats_body, nf),
            grid=(grid,),
            in_specs=[part_spec] * 3 + [full_spec, bias_spec],
            out_specs=[row_spec] * 3 + [stats_spec],
            out_shape=[row_sd] * 3 + [stats_sd],
        )(parts[0], parts[1], parts[2], wmat, bias.reshape(1, h))
        return z_and_stats[:3], z_and_stats[3]

    z2, stats2 = dense_hop(h1, W0, b0)
    h2 = pl.pallas_call(
        functools.partial(_apply_bn_body, nf),
        grid=(grid,),
        in_specs=[row_spec] * 3 + [stats_spec],
        out_specs=[row_spec] * 3,
        out_shape=[row_sd] * 3,
    )(z2[0], z2[1], z2[2], stats2)

    z3, stats3 = dense_hop(h2, W1, b1)

    h_final, nsum = pl.pallas_call(
        functools.partial(_final_body, nf),
        grid=(grid,),
        in_specs=[row_spec] * 9 + [stats_spec],
        out_specs=[out_spec, scal_spec],
        out_shape=[jax.ShapeDtypeStruct((n, 6 * h), F32),
                   jax.ShapeDtypeStruct((1, 1), F32)],
    )(h1[0], h1[1], h1[2], h2[0], h2[1], h2[2],
      z3[0], z3[1], z3[2], stats3)

    t_norm = nsum[0, 0] / nf
    final_loss = (1.0 / 3.0) / jnp.log(t_norm + E_CONSTANT)
    return (h_final, final_loss)


# speedup vs baseline: 1.0563x; 1.0563x over previous
import functools

import jax
import jax.numpy as jnp
from jax import lax
from jax.experimental import pallas as pl
from jax.experimental.pallas import tpu as pltpu
from jax.experimental.pallas import tpu_sc as plsc

R = 3
E_CONSTANT = 2.7182817459106445
BN_EPS = 1e-5
F32 = jnp.float32

_NC = 2
_NS = 16
_CH = 120


def _mm_stats_body(n, x_ref, age_ref, w_ref, y_ref, stats_ref):
    i = pl.program_id(0)
    y = jnp.dot(x_ref[...], w_ref[...], preferred_element_type=F32)
    y_ref[...] = y
    age = age_ref[...].astype(F32)
    rows = []
    for w in range(R):
        m = (age >= float(w)).astype(F32)
        ym = y * m
        rows.append(jnp.sum(ym, axis=0))
        rows.append(jnp.sum(ym * ym, axis=0))
    z = jnp.zeros_like(rows[0])
    stats = jnp.stack(rows + [z, z])

    @pl.when(i == 0)
    def _():
        stats_ref[...] = stats

    @pl.when(i != 0)
    def _():
        stats_ref[...] += stats


def _apply1_body(n, y_ref, age_ref, stats_ref, h0_ref, h1_ref, h2_ref):
    y = y_ref[...]
    age = age_ref[...].astype(F32)
    stats = stats_ref[...]
    outs = (h0_ref, h1_ref, h2_ref)
    for w in range(R):
        s = stats[2 * w]
        q = stats[2 * w + 1]
        m = s / n
        inv = lax.rsqrt(q / n - m * m + BN_EPS)
        msk = (age >= float(w)).astype(F32)
        outs[w][...] = jnp.maximum((y * msk - m[None, :]) * inv[None, :], 0.0)


def _z_stats_body(n, p0_ref, p1_ref, p2_ref, w_ref, b_ref,
                  z0_ref, z1_ref, z2_ref, stats_ref):
    i = pl.program_id(0)
    wmat = w_ref[...]
    b = b_ref[...]
    rows = []
    for p_ref, z_ref in ((p0_ref, z0_ref), (p1_ref, z1_ref), (p2_ref, z2_ref)):
        agg = p_ref[0] + p_ref[1]
        z = jnp.dot(agg, wmat, preferred_element_type=F32) + b
        z_ref[...] = z
        rows.append(jnp.sum(z, axis=0))
        rows.append(jnp.sum(z * z, axis=0))
    zr = jnp.zeros_like(rows[0])
    stats = jnp.stack(rows + [zr, zr])

    @pl.when(i == 0)
    def _():
        stats_ref[...] = stats

    @pl.when(i != 0)
    def _():
        stats_ref[...] += stats


def _apply_bn_body(n, z0_ref, z1_ref, z2_ref, stats_ref, h0_ref, h1_ref, h2_ref):
    stats = stats_ref[...]
    zs = (z0_ref, z1_ref, z2_ref)
    outs = (h0_ref, h1_ref, h2_ref)
    for w in range(R):
        s = stats[2 * w]
        q = stats[2 * w + 1]
        m = s / n
        inv = lax.rsqrt(q / n - m * m + BN_EPS)
        outs[w][...] = jnp.maximum((zs[w][...] - m[None, :]) * inv[None, :], 0.0)


def _final_body(n, h1_0, h1_1, h1_2, h2_0, h2_1, h2_2,
                z3_0, z3_1, z3_2, stats_ref, out_ref, nsum_ref):
    i = pl.program_id(0)
    stats = stats_ref[...]
    h1s = (h1_0, h1_1, h1_2)
    h2s = (h2_0, h2_1, h2_2)
    z3s = (z3_0, z3_1, z3_2)
    wins = []
    for w in range(R):
        s = stats[2 * w]
        q = stats[2 * w + 1]
        m = s / n
        inv = lax.rsqrt(q / n - m * m + BN_EPS)
        h3 = jnp.maximum((z3s[w][...] - m[None, :]) * inv[None, :], 0.0)
        wins.append((h1s[w][...], h2s[w][...], h3))
    normsq = None
    for j in range(3):
        a0 = wins[0][j]
        t = a0 - 0.5 * (wins[1][j] + wins[2][j])
        out_ref[:, j * 128:(j + 1) * 128] = a0
        out_ref[:, 384 + j * 128:384 + (j + 1) * 128] = t
        nsq = jnp.sum(t * t, axis=1)
        normsq = nsq if normsq is None else normsq + nsq
    total = jnp.sum(jnp.sqrt(normsq)).reshape(1, 1)

    @pl.when(i == 0)
    def _():
        nsum_ref[...] = total

    @pl.when(i != 0)
    def _():
        nsum_ref[...] += total


def _make_hop(n_nodes, e_pad, r_acc, nch0, nch1):
    rows_per_tile = r_acc // _NS
    mesh = plsc.VectorSubcoreMesh(core_axis_name="c", subcore_axis_name="s")
    out_sd = jax.ShapeDtypeStruct((_NC, r_acc, 128), F32)

    @functools.partial(
        pl.kernel,
        mesh=mesh,
        out_type=(out_sd, out_sd, out_sd),
        scratch_types=[
            pltpu.VMEM_SHARED((r_acc, 128), F32),
            pltpu.VMEM((6, 2, _CH), jnp.int32),
            pltpu.VMEM((3, _CH, 128), F32),
            pltpu.SemaphoreType.DMA((3,)),
            pltpu.SemaphoreType.DMA((3,)),
            pltpu.SemaphoreType.DMA((6,)),
        ],
    )
    def hop(t0, t1, t2, idx_hbm, o0, o1, o2,
            acc, idxb, rowb, gsem, ssem, isem):
        c = lax.axis_index("c")
        s = lax.axis_index("s")
        tile_row0 = s * rows_per_tile
        nch = jnp.where(c == 0, nch0, nch1)
        chunk0 = jnp.where(c == 0, s * nch0, _NS * nch0 + s * nch1)

        def _ld_idx(i, slot):
            return pltpu.make_async_copy(idx_hbm.at[chunk0 + i],
                                         idxb.at[slot], isem.at[slot])

        def _gather(tbl, i6, b3):
            return pltpu.make_async_copy(tbl.at[idxb.at[i6, 0]],
                                         rowb.at[b3], gsem.at[b3])

        def _scatter(i6, b3):
            return pltpu.make_async_copy(rowb.at[b3],
                                         acc.at[idxb.at[i6, 1]],
                                         ssem.at[b3])

        for w in range(R):
            tbl = (t0, t1, t2)[w]
            out = (o0, o1, o2)[w]

            def _zb(i, carry):
                for j in range(128 // 16):
                    rowb[0, i, pl.ds(j * 16, 16)] = jnp.zeros((16,), F32)
                return carry

            lax.fori_loop(0, _CH, _zb, 0)
            done = 0
            while done < rows_per_tile:
                rows = min(_CH, rows_per_tile - done)
                pltpu.sync_copy(rowb.at[0, pl.ds(0, rows)],
                                acc.at[pl.ds(tile_row0 + done, rows)])
                done += rows
            plsc.subcore_barrier()

            for k in range(4):
                _ld_idx(k, k).start()
            _ld_idx(0, 0).wait()
            _ld_idx(1, 1).wait()
            _gather(tbl, 0, 0).start()
            _gather(tbl, 1, 1).start()

            def _grp(g, carry):
                for b in range(6):
                    i = 6 * g + b
                    b3 = b % 3
                    _gather(tbl, b, b3).wait()

                    @pl.when(i >= 1)
                    def _():
                        _scatter((b + 5) % 6, (b3 + 2) % 3).wait()

                    _scatter(b, b3).start(add=True)

                    @pl.when(i + 4 < nch)
                    def _():
                        _ld_idx(i + 4, (b + 4) % 6).start()

                    @pl.when(i + 2 < nch)
                    def _():
                        _ld_idx(i + 2, (b + 2) % 6).wait()
                        _gather(tbl, (b + 2) % 6, (b3 + 2) % 3).start()
                return carry

            lax.fori_loop(0, nch // 6, _grp, 0)
            _scatter(5, 2).wait()
            plsc.subcore_barrier()

            pltpu.sync_copy(acc.at[pl.ds(tile_row0, rows_per_tile)],
                            out.at[c, pl.ds(tile_row0, rows_per_tile)])
            plsc.subcore_barrier()

    return hop


def _cdiv(a, b):
    return (a + b - 1) // b


def kernel(x, edge_index, age, init_weight, W0, b0, W1, b1):
    n, d = x.shape
    h = init_weight.shape[1]
    e = edge_index.shape[1]
    bn = 1000
    grid = n // bn
    nf = float(n)

    e_unit = _NC * _NS * 6 * _CH
    e_pad = _cdiv(e, e_unit) * e_unit
    r_acc = _cdiv(n, _NS * 8) * (_NS * 8)
    if r_acc == n:
        r_acc += _NS * 8
    pad = e_pad - e
    dummy = n + jnp.arange(pad, dtype=jnp.int32) % (r_acc - n)
    src = jnp.concatenate([edge_index[0], jnp.zeros((pad,), jnp.int32)])
    dst = jnp.concatenate([edge_index[1], dummy])
    idx3 = jnp.stack([src.reshape(-1, _CH), dst.reshape(-1, _CH)], axis=1)
    age2 = age.reshape(n, 1)

    row_spec = pl.BlockSpec((bn, h), lambda i: (i, 0))
    age_spec = pl.BlockSpec((bn, 1), lambda i: (i, 0))
    stats_spec = pl.BlockSpec((8, h), lambda i: (0, 0))
    full_spec = pl.BlockSpec((d, h), lambda i: (0, 0))
    bias_spec = pl.BlockSpec((1, h), lambda i: (0, 0))
    part_spec = pl.BlockSpec((_NC, bn, h), lambda i: (0, i, 0))
    out_spec = pl.BlockSpec((bn, 6 * h), lambda i: (i, 0))
    scal_spec = pl.BlockSpec((1, 1), lambda i: (0, 0))

    row_sd = jax.ShapeDtypeStruct((n, h), F32)
    stats_sd = jax.ShapeDtypeStruct((8, h), F32)

    y, stats1 = pl.pallas_call(
        functools.partial(_mm_stats_body, nf),
        grid=(grid,),
        in_specs=[row_spec, age_spec, full_spec],
        out_specs=[row_spec, stats_spec],
        out_shape=[row_sd, stats_sd],
    )(x, age2, init_weight)

    h1 = pl.pallas_call(
        functools.partial(_apply1_body, nf),
        grid=(grid,),
        in_specs=[row_spec, age_spec, stats_spec],
        out_specs=[row_spec] * 3,
        out_shape=[row_sd] * 3,
    )(y, age2, stats1)

    nch_tot = e_pad // (_NS * _CH)
    nch0 = (int(round(nch_tot * 0.928)) // 6) * 6
    nch1 = nch_tot - nch0
    hop = _make_hop(n, e_pad, r_acc, nch0, nch1)

    def dense_hop(tables, wmat, bias):
        parts = hop(tables[0], tables[1], tables[2], idx3)
        z_and_stats = pl.pallas_call(
            functools.partial(_z_stats_body, nf),
            grid=(grid,),
            in_specs=[part_spec] * 3 + [full_spec, bias_spec],
            out_specs=[row_spec] * 3 + [stats_spec],
            out_shape=[row_sd] * 3 + [stats_sd],
        )(parts[0], parts[1], parts[2], wmat, bias.reshape(1, h))
        return z_and_stats[:3], z_and_stats[3]

    z2, stats2 = dense_hop(h1, W0, b0)
    h2 = pl.pallas_call(
        functools.partial(_apply_bn_body, nf),
        grid=(grid,),
        in_specs=[row_spec] * 3 + [stats_spec],
        out_specs=[row_spec] * 3,
        out_shape=[row_sd] * 3,
    )(z2[0], z2[1], z2[2], stats2)

    z3, stats3 = dense_hop(h2, W1, b1)

    h_final, nsum = pl.pallas_call(
        functools.partial(_final_body, nf),
        grid=(grid,),
        in_specs=[row_spec] * 9 + [stats_spec],
        out_specs=[out_spec, scal_spec],
        out_shape=[jax.ShapeDtypeStruct((n, 6 * h), F32),
                   jax.ShapeDtypeStruct((1, 1), F32)],
    )(h1[0], h1[1], h1[2], h2[0], h2[1], h2[2],
      z3[0], z3[1], z3[2], stats3)

    t_norm = nsum[0, 0] / nf
    final_loss = (1.0 / 3.0) / jnp.log(t_norm + E_CONSTANT)
    return (h_final, final_loss)
